# baseline (speedup 1.0000x reference)
import functools

import jax
import jax.numpy as jnp
from jax import lax
from jax.experimental import pallas as pl
from jax.experimental.pallas import tpu as pltpu
from jax.experimental.pallas import tpu_sc as plsc

BLK = 256


def _sc_dispatch_gather(table, idx):
    info = plsc.get_sparse_core_info()
    nc, ns = info.num_cores, info.num_subcores
    nw = nc * ns
    v, d = table.shape
    b = idx.shape[0]
    b_per_w = b // nw
    chunk = min(b_per_w, max(8, (256 * 1024) // (d * 4)))
    n_chunks = b_per_w // chunk
    mesh = plsc.VectorSubcoreMesh(core_axis_name="c", subcore_axis_name="s")

    @functools.partial(
        pl.kernel, mesh=mesh,
        out_type=jax.ShapeDtypeStruct((b, d), jnp.float32),
        scratch_types=[
            pltpu.VMEM((chunk,), jnp.int32),
            pltpu.VMEM((chunk, d), jnp.float32),
            pltpu.SemaphoreType.DMA,
        ],
    )
    def k(table_hbm, idx_hbm, out_hbm, idx_v, rows_v, sem):
        wid = lax.axis_index("s") * nc + lax.axis_index("c")
        for c in range(n_chunks):
            base = wid * b_per_w + c * chunk
            pltpu.sync_copy(idx_hbm.at[pl.ds(base, chunk)], idx_v)
            pltpu.async_copy(table_hbm.at[idx_v], rows_v, sem).wait()
            pltpu.sync_copy(rows_v, out_hbm.at[pl.ds(base, chunk)])

    return k(table, idx)


def _moe_tile_kernel(te_ref, tb_ref, ts_ref, tn_ref,
                     x_ref, gate_ref, up_ref, down_ref,
                     y_ref):
    t = pl.program_id(0)
    start = ts_ref[t]
    end = tn_ref[t]
    b = tb_ref[t]

    @pl.when(end > start)
    def _():
        xb = x_ref[...]
        g = jax.lax.dot_general(
            xb, gate_ref[0], (((1,), (1,)), ((), ())),
            preferred_element_type=jnp.float32)
        u = jax.lax.dot_general(
            xb, up_ref[0], (((1,), (1,)), ((), ())),
            preferred_element_type=jnp.float32)
        inter = g * jax.nn.sigmoid(g) * u
        y = jax.lax.dot_general(
            inter, down_ref[0], (((1,), (1,)), ((), ())),
            preferred_element_type=jnp.float32)
        rows = b * BLK + jax.lax.broadcasted_iota(jnp.int32, (BLK, 1), 0)
        mask = (rows >= start) & (rows < end)
        y_ref[...] = jnp.where(mask, y, y_ref[...])


def kernel(x, expert_indices, expert_weights, gate_proj, up_proj, down_proj):
    batch, seq, hidden = x.shape
    num_experts, ff, _ = gate_proj.shape
    top_k = expert_indices.shape[-1]
    num_tokens = batch * seq
    num_assign = num_tokens * top_k

    x_flat = x.reshape(num_tokens, hidden)
    e_flat = expert_indices.reshape(-1).astype(jnp.int32)

    onehot = (e_flat[:, None] == jnp.arange(num_experts, dtype=jnp.int32)[None, :])
    csum = jnp.cumsum(onehot.astype(jnp.int32), axis=0)
    counts = csum[-1]
    off = jnp.concatenate([jnp.zeros(1, jnp.int32),
                           jnp.cumsum(counts, dtype=jnp.int32)])
    rank = jnp.take_along_axis(csum, e_flat[:, None], axis=1)[:, 0] - 1
    pos = off[e_flat] + rank
    token_assign = jnp.arange(num_assign, dtype=jnp.int32) // top_k
    sorted_token = jnp.zeros(num_assign, jnp.int32).at[pos].set(token_assign)

    nb = num_assign // BLK
    T = nb + num_experts - 1
    cnt = off[1:] - off[:-1]
    fb = off[:-1] // BLK
    lb = jnp.where(cnt > 0, (off[1:] - 1) // BLK, fb - 1)
    nbe = jnp.maximum(lb - fb + 1, 0)
    tstart = jnp.concatenate([jnp.zeros(1, jnp.int32),
                              jnp.cumsum(nbe, dtype=jnp.int32)])
    t_actual = tstart[num_experts]
    t_ids = jnp.arange(T, dtype=jnp.int32)
    t_eff = jnp.minimum(t_ids, t_actual - 1)
    e_of_t = (jnp.searchsorted(tstart, t_eff, side='right') - 1).astype(jnp.int32)
    b_of_t = fb[e_of_t] + (t_eff - tstart[e_of_t])
    s_of_t = jnp.maximum(off[e_of_t], b_of_t * BLK)
    n_of_t = jnp.minimum(off[e_of_t + 1], (b_of_t + 1) * BLK)
    active = t_ids < t_actual
    s_of_t = jnp.where(active, s_of_t, 0)
    n_of_t = jnp.where(active, n_of_t, 0)

    x_sorted = _sc_dispatch_gather(x_flat, sorted_token)

    grid_spec = pltpu.PrefetchScalarGridSpec(
        num_scalar_prefetch=4,
        grid=(T,),
        in_specs=[
            pl.BlockSpec((BLK, hidden), lambda t, te, tb, ts, tn: (tb[t], 0)),
            pl.BlockSpec((1, ff, hidden), lambda t, te, tb, ts, tn: (te[t], 0, 0)),
            pl.BlockSpec((1, ff, hidden), lambda t, te, tb, ts, tn: (te[t], 0, 0)),
            pl.BlockSpec((1, hidden, ff), lambda t, te, tb, ts, tn: (te[t], 0, 0)),
        ],
        out_specs=pl.BlockSpec((BLK, hidden), lambda t, te, tb, ts, tn: (tb[t], 0)),
    )
    y_sorted = pl.pallas_call(
        _moe_tile_kernel,
        grid_spec=grid_spec,
        out_shape=jax.ShapeDtypeStruct((num_assign, hidden), jnp.float32),
    )(e_of_t, b_of_t, s_of_t, n_of_t,
      x_sorted, gate_proj, up_proj, down_proj)

    y_unsorted = _sc_dispatch_gather(y_sorted, pos).reshape(
        num_tokens, top_k, hidden)
    w2 = expert_weights.reshape(num_tokens, top_k)
    out = (w2[:, 0:1] * y_unsorted[:, 0, :] + w2[:, 1:2] * y_unsorted[:, 1, :])
    return out.reshape(batch, seq, hidden)


# speedup vs baseline: 1.0073x; 1.0073x over previous
import functools

import jax
import jax.numpy as jnp
from jax import lax
from jax.experimental import pallas as pl
from jax.experimental.pallas import tpu as pltpu
from jax.experimental.pallas import tpu_sc as plsc

BLK = 256


def _sc_dispatch_gather(table, idx):
    info = plsc.get_sparse_core_info()
    nc, ns = info.num_cores, info.num_subcores
    nw = nc * ns
    v, d = table.shape
    b = idx.shape[0]
    b_per_w = b // nw
    chunk = min(b_per_w, max(8, (256 * 1024) // (d * 4)))
    n_chunks = b_per_w // chunk
    mesh = plsc.VectorSubcoreMesh(core_axis_name="c", subcore_axis_name="s")

    @functools.partial(
        pl.kernel, mesh=mesh,
        out_type=jax.ShapeDtypeStruct((b, d), jnp.float32),
        scratch_types=[
            pltpu.VMEM((chunk,), jnp.int32),
            pltpu.VMEM((chunk, d), jnp.float32),
            pltpu.SemaphoreType.DMA,
        ],
    )
    def k(table_hbm, idx_hbm, out_hbm, idx_v, rows_v, sem):
        wid = lax.axis_index("s") * nc + lax.axis_index("c")
        for c in range(n_chunks):
            base = wid * b_per_w + c * chunk
            pltpu.sync_copy(idx_hbm.at[pl.ds(base, chunk)], idx_v)
            pltpu.async_copy(table_hbm.at[idx_v], rows_v, sem).wait()
            pltpu.sync_copy(rows_v, out_hbm.at[pl.ds(base, chunk)])

    return k(table, idx)


def _moe_tile_kernel(te_ref, tb_ref, ts_ref, tn_ref,
                     x_ref, gate_ref, up_ref, down_ref,
                     y_ref):
    t = pl.program_id(0)
    start = ts_ref[t]
    end = tn_ref[t]
    b = tb_ref[t]

    xb = x_ref[...]
    g = jax.lax.dot_general(
        xb, gate_ref[0], (((1,), (1,)), ((), ())),
        preferred_element_type=jnp.float32)
    u = jax.lax.dot_general(
        xb, up_ref[0], (((1,), (1,)), ((), ())),
        preferred_element_type=jnp.float32)
    inter = g * jax.nn.sigmoid(g) * u
    y = jax.lax.dot_general(
        inter, down_ref[0], (((1,), (1,)), ((), ())),
        preferred_element_type=jnp.float32)
    rows = b * BLK + jax.lax.broadcasted_iota(jnp.int32, (BLK, 1), 0)
    mask = (rows >= start) & (rows < end)
    y_ref[...] = jnp.where(mask, y, y_ref[...])


def kernel(x, expert_indices, expert_weights, gate_proj, up_proj, down_proj):
    batch, seq, hidden = x.shape
    num_experts, ff, _ = gate_proj.shape
    top_k = expert_indices.shape[-1]
    num_tokens = batch * seq
    num_assign = num_tokens * top_k

    x_flat = x.reshape(num_tokens, hidden)
    e_flat = expert_indices.reshape(-1).astype(jnp.int32)

    onehot = (e_flat[:, None] == jnp.arange(num_experts, dtype=jnp.int32)[None, :])
    csum = jnp.cumsum(onehot.astype(jnp.int32), axis=0)
    counts = csum[-1]
    off = jnp.concatenate([jnp.zeros(1, jnp.int32),
                           jnp.cumsum(counts, dtype=jnp.int32)])
    rank = jnp.take_along_axis(csum, e_flat[:, None], axis=1)[:, 0] - 1
    pos = off[e_flat] + rank
    token_assign = jnp.arange(num_assign, dtype=jnp.int32) // top_k
    sorted_token = jnp.zeros(num_assign, jnp.int32).at[pos].set(token_assign)

    nb = num_assign // BLK
    T = nb + num_experts - 1
    cnt = off[1:] - off[:-1]
    fb = off[:-1] // BLK
    lb = jnp.where(cnt > 0, (off[1:] - 1) // BLK, fb - 1)
    nbe = jnp.maximum(lb - fb + 1, 0)
    tstart = jnp.concatenate([jnp.zeros(1, jnp.int32),
                              jnp.cumsum(nbe, dtype=jnp.int32)])
    t_actual = tstart[num_experts]
    t_ids = jnp.arange(T, dtype=jnp.int32)
    t_eff = jnp.minimum(t_ids, t_actual - 1)
    e_of_t = (jnp.searchsorted(tstart, t_eff, side='right') - 1).astype(jnp.int32)
    b_of_t = fb[e_of_t] + (t_eff - tstart[e_of_t])
    s_of_t = jnp.maximum(off[e_of_t], b_of_t * BLK)
    n_of_t = jnp.minimum(off[e_of_t + 1], (b_of_t + 1) * BLK)
    active = t_ids < t_actual
    s_of_t = jnp.where(active, s_of_t, 0)
    n_of_t = jnp.where(active, n_of_t, 0)

    x_sorted = _sc_dispatch_gather(x_flat, sorted_token)

    grid_spec = pltpu.PrefetchScalarGridSpec(
        num_scalar_prefetch=4,
        grid=(T,),
        in_specs=[
            pl.BlockSpec((BLK, hidden), lambda t, te, tb, ts, tn: (tb[t], 0)),
            pl.BlockSpec((1, ff, hidden), lambda t, te, tb, ts, tn: (te[t], 0, 0)),
            pl.BlockSpec((1, ff, hidden), lambda t, te, tb, ts, tn: (te[t], 0, 0)),
            pl.BlockSpec((1, hidden, ff), lambda t, te, tb, ts, tn: (te[t], 0, 0)),
        ],
        out_specs=pl.BlockSpec((BLK, hidden), lambda t, te, tb, ts, tn: (tb[t], 0)),
    )
    y_sorted = pl.pallas_call(
        _moe_tile_kernel,
        grid_spec=grid_spec,
        out_shape=jax.ShapeDtypeStruct((num_assign, hidden), jnp.float32),
    )(e_of_t, b_of_t, s_of_t, n_of_t,
      x_sorted, gate_proj, up_proj, down_proj)

    y_unsorted = y_sorted[pos].reshape(num_tokens, top_k, hidden)
    w2 = expert_weights.reshape(num_tokens, top_k)
    out = (w2[:, 0:1] * y_unsorted[:, 0, :] + w2[:, 1:2] * y_unsorted[:, 1, :])
    return out.reshape(batch, seq, hidden)


# speedup vs baseline: 1.0136x; 1.0063x over previous
import functools

import jax
import jax.numpy as jnp
from jax import lax
from jax.experimental import pallas as pl
from jax.experimental.pallas import tpu as pltpu
from jax.experimental.pallas import tpu_sc as plsc

BLK = 256


def _sc_dispatch_gather(table, idx):
    info = plsc.get_sparse_core_info()
    nc, ns = info.num_cores, info.num_subcores
    nw = nc * ns
    v, d = table.shape
    b = idx.shape[0]
    b_per_w = b // nw
    chunk = min(b_per_w, max(8, (256 * 1024) // (d * 4)))
    n_chunks = b_per_w // chunk
    mesh = plsc.VectorSubcoreMesh(core_axis_name="c", subcore_axis_name="s")

    @functools.partial(
        pl.kernel, mesh=mesh,
        out_type=jax.ShapeDtypeStruct((b, d), jnp.float32),
        scratch_types=[
            pltpu.VMEM((chunk,), jnp.int32),
            pltpu.VMEM((chunk, d), jnp.float32),
            pltpu.SemaphoreType.DMA,
        ],
    )
    def k(table_hbm, idx_hbm, out_hbm, idx_v, rows_v, sem):
        wid = lax.axis_index("s") * nc + lax.axis_index("c")
        for c in range(n_chunks):
            base = wid * b_per_w + c * chunk
            pltpu.sync_copy(idx_hbm.at[pl.ds(base, chunk)], idx_v)
            pltpu.async_copy(table_hbm.at[idx_v], rows_v, sem).wait()
            pltpu.sync_copy(rows_v, out_hbm.at[pl.ds(base, chunk)])

    return k(table, idx)


def _moe_tile_kernel(te_ref, tb_ref, ts_ref, tn_ref,
                     x_ref, gate_ref, up_ref, down_ref,
                     y_ref):
    t = pl.program_id(0)
    start = ts_ref[t]
    end = tn_ref[t]
    b = tb_ref[t]

    xb = x_ref[...]
    g = jax.lax.dot_general(
        xb, gate_ref[0], (((1,), (1,)), ((), ())),
        preferred_element_type=jnp.float32)
    u = jax.lax.dot_general(
        xb, up_ref[0], (((1,), (1,)), ((), ())),
        preferred_element_type=jnp.float32)
    inter = g * jax.nn.sigmoid(g) * u
    y = jax.lax.dot_general(
        inter, down_ref[0], (((1,), (1,)), ((), ())),
        preferred_element_type=jnp.float32)
    rows = b * BLK + jax.lax.broadcasted_iota(jnp.int32, (BLK, 1), 0)
    mask = (rows >= start) & (rows < end)
    y_ref[...] = jnp.where(mask, y, y_ref[...])


def kernel(x, expert_indices, expert_weights, gate_proj, up_proj, down_proj):
    batch, seq, hidden = x.shape
    num_experts, ff, _ = gate_proj.shape
    top_k = expert_indices.shape[-1]
    num_tokens = batch * seq
    num_assign = num_tokens * top_k

    x_flat = x.reshape(num_tokens, hidden)
    e_flat = expert_indices.reshape(-1).astype(jnp.int32)

    onehot = (e_flat[:, None] == jnp.arange(num_experts, dtype=jnp.int32)[None, :])
    oh3 = onehot.astype(jnp.int32).reshape(32, num_assign // 32, num_experts)
    c1 = jnp.cumsum(oh3, axis=1)
    rowtot = c1[:, -1, :]
    rowoff = jnp.cumsum(rowtot, axis=0) - rowtot
    csum = (c1 + rowoff[:, None, :]).reshape(num_assign, num_experts)
    counts = csum[-1]
    off = jnp.concatenate([jnp.zeros(1, jnp.int32),
                           jnp.cumsum(counts, dtype=jnp.int32)])
    rank = jnp.take_along_axis(csum, e_flat[:, None], axis=1)[:, 0] - 1
    pos = off[e_flat] + rank
    token_assign = jnp.arange(num_assign, dtype=jnp.int32) // top_k
    sorted_token = jnp.zeros(num_assign, jnp.int32).at[pos].set(token_assign)

    nb = num_assign // BLK
    T = nb + num_experts - 1
    cnt = off[1:] - off[:-1]
    fb = off[:-1] // BLK
    lb = jnp.where(cnt > 0, (off[1:] - 1) // BLK, fb - 1)
    nbe = jnp.maximum(lb - fb + 1, 0)
    tstart = jnp.concatenate([jnp.zeros(1, jnp.int32),
                              jnp.cumsum(nbe, dtype=jnp.int32)])
    t_actual = tstart[num_experts]
    t_ids = jnp.arange(T, dtype=jnp.int32)
    t_eff = jnp.minimum(t_ids, t_actual - 1)
    e_of_t = (jnp.searchsorted(tstart, t_eff, side='right') - 1).astype(jnp.int32)
    b_of_t = fb[e_of_t] + (t_eff - tstart[e_of_t])
    s_of_t = jnp.maximum(off[e_of_t], b_of_t * BLK)
    n_of_t = jnp.minimum(off[e_of_t + 1], (b_of_t + 1) * BLK)
    active = t_ids < t_actual
    s_of_t = jnp.where(active, s_of_t, 0)
    n_of_t = jnp.where(active, n_of_t, 0)

    x_sorted = _sc_dispatch_gather(x_flat, sorted_token)

    grid_spec = pltpu.PrefetchScalarGridSpec(
        num_scalar_prefetch=4,
        grid=(T,),
        in_specs=[
            pl.BlockSpec((BLK, hidden), lambda t, te, tb, ts, tn: (tb[t], 0)),
            pl.BlockSpec((1, ff, hidden), lambda t, te, tb, ts, tn: (te[t], 0, 0)),
            pl.BlockSpec((1, ff, hidden), lambda t, te, tb, ts, tn: (te[t], 0, 0)),
            pl.BlockSpec((1, hidden, ff), lambda t, te, tb, ts, tn: (te[t], 0, 0)),
        ],
        out_specs=pl.BlockSpec((BLK, hidden), lambda t, te, tb, ts, tn: (tb[t], 0)),
    )
    y_sorted = pl.pallas_call(
        _moe_tile_kernel,
        grid_spec=grid_spec,
        out_shape=jax.ShapeDtypeStruct((num_assign, hidden), jnp.float32),
    )(e_of_t, b_of_t, s_of_t, n_of_t,
      x_sorted, gate_proj, up_proj, down_proj)

    y_unsorted = y_sorted[pos].reshape(num_tokens, top_k, hidden)
    w2 = expert_weights.reshape(num_tokens, top_k)
    out = (w2[:, 0:1] * y_unsorted[:, 0, :] + w2[:, 1:2] * y_unsorted[:, 1, :])
    return out.reshape(batch, seq, hidden)
